# baseline (speedup 1.0000x reference)
import functools

import jax
import jax.numpy as jnp
from jax import lax
from jax.experimental import pallas as pl
from jax.experimental.pallas import tpu as pltpu
from jax.experimental.pallas import tpu_sc as plsc

_info = plsc.get_sparse_core_info()
_NC = _info.num_cores
_NS = _info.num_subcores
_L = _info.num_lanes
_NW = _NC * _NS


@functools.lru_cache(maxsize=None)
def _make_gather(B, V, D, chunk, nbuf):
    bpw = B // _NW
    sizes = []
    left = bpw
    while left > 0:
        sizes.append(min(chunk, left))
        left -= sizes[-1]
    offs = [sum(sizes[:i]) for i in range(len(sizes))]
    n = len(sizes)
    mesh = plsc.VectorSubcoreMesh(core_axis_name="c", subcore_axis_name="s")

    scratch = [
        pltpu.VMEM((bpw,), jnp.int32),
        pltpu.VMEM((_L,), jnp.int32),
    ]
    scratch += [pltpu.VMEM((chunk, D), jnp.float32) for _ in range(nbuf)]
    scratch += [pltpu.SemaphoreType.DMA for _ in range(2 * nbuf)]

    @functools.partial(
        pl.kernel,
        mesh=mesh,
        out_type=jax.ShapeDtypeStruct((B, 1, D), jnp.float32),
        scratch_types=scratch,
    )
    def k(table_hbm, ids_hbm, gid_hbm, out_hbm, idx_v, g_v, *bufs_sems):
        bufs = bufs_sems[:nbuf]
        gsems = bufs_sems[nbuf:2 * nbuf]
        wsems = bufs_sems[2 * nbuf:]
        wid = lax.axis_index("s") * _NC + lax.axis_index("c")
        base = wid * bpw
        cp_ids = pltpu.async_copy(ids_hbm.at[pl.ds(base, bpw)], idx_v,
                                  gsems[0])
        cp_gid = pltpu.async_copy(gid_hbm, g_v.at[pl.ds(0, 1)], gsems[1])
        cp_gid.wait()
        gofs = lax.gather(
            g_v[...], jnp.zeros((_L, 1), jnp.int32),
            lax.GatherDimensionNumbers(offset_dims=(),
                                       collapsed_slice_dims=(0,),
                                       start_index_map=(0,)),
            (1,), mode=lax.GatherScatterMode.PROMISE_IN_BOUNDS) * V
        cp_ids.wait()

        def add_offsets(c):
            for i in range(offs[c] // _L, (offs[c] + sizes[c]) // _L):
                sl = pl.ds(i * _L, _L)
                idx_v[sl] = idx_v[sl] + gofs

        def gather(c):
            return pltpu.async_copy(
                table_hbm.at[idx_v.at[pl.ds(offs[c], sizes[c])]],
                bufs[c % nbuf].at[pl.ds(0, sizes[c])], gsems[c % nbuf])

        def write(c):
            return pltpu.async_copy(
                bufs[c % nbuf].at[pl.ds(0, sizes[c])],
                out_hbm.at[pl.ds(base + offs[c], sizes[c]), 0],
                wsems[c % nbuf])

        gath = [None] * n
        wr = [None] * n
        head = min(nbuf, n)
        for c in range(head):
            add_offsets(c)
            gath[c] = gather(c)
        for c in range(head, n):
            add_offsets(c)
        drained = [False] * n
        for c in range(n):
            gath[c].wait()
            wr[c] = write(c)
            if c + nbuf < n:
                wr[c].wait()
                drained[c] = True
                gath[c + nbuf] = gather(c + nbuf)
        for c in range(n):
            if not drained[c]:
                wr[c].wait()

    return k


def kernel(tables, token_ids, group_idx):
    G, V, D = tables.shape
    B, S = token_ids.shape
    table_flat = tables.reshape(G * V, D)
    ids = token_ids.reshape(B * S)
    gid = jnp.asarray(group_idx, jnp.int32).reshape(1)
    out = _make_gather(B * S, V, D, 16, 7)(table_flat, ids, gid)
    return out.reshape(B, S, D) if S != 1 else out


# speedup vs baseline: 1.0138x; 1.0138x over previous
import functools

import jax
import jax.numpy as jnp
from jax import lax
from jax.experimental import pallas as pl
from jax.experimental.pallas import tpu as pltpu
from jax.experimental.pallas import tpu_sc as plsc

_info = plsc.get_sparse_core_info()
_NC = _info.num_cores
_NS = _info.num_subcores
_L = _info.num_lanes
_NW = _NC * _NS


@functools.lru_cache(maxsize=None)
def _make_gather(B, V, D, chunk, nbuf):
    bpw = B // _NW
    sizes = []
    left = bpw
    while left > 0:
        sizes.append(min(chunk, left))
        left -= sizes[-1]
    offs = [sum(sizes[:i]) for i in range(len(sizes))]
    n = len(sizes)
    mesh = plsc.VectorSubcoreMesh(core_axis_name="c", subcore_axis_name="s")

    scratch = [
        pltpu.VMEM((bpw,), jnp.int32),
        pltpu.VMEM((_L,), jnp.int32),
    ]
    scratch += [pltpu.VMEM((chunk, D), jnp.float32) for _ in range(nbuf)]
    scratch += [pltpu.SemaphoreType.DMA for _ in range(2 * nbuf)]

    @functools.partial(
        pl.kernel,
        mesh=mesh,
        out_type=jax.ShapeDtypeStruct((B, 1, D), jnp.float32),
        scratch_types=scratch,
    )
    def k(table_hbm, ids_hbm, gid_hbm, out_hbm, idx_v, g_v, *bufs_sems):
        bufs = bufs_sems[:nbuf]
        gsems = bufs_sems[nbuf:2 * nbuf]
        wsems = bufs_sems[2 * nbuf:]
        wid = lax.axis_index("s") * _NC + lax.axis_index("c")
        base = wid * bpw
        cp_ids = pltpu.async_copy(ids_hbm.at[pl.ds(base, bpw)], idx_v,
                                  gsems[0])
        cp_gid = pltpu.async_copy(gid_hbm, g_v.at[pl.ds(0, 1)], gsems[1])
        cp_gid.wait()
        gofs = lax.gather(
            g_v[...], jnp.zeros((_L, 1), jnp.int32),
            lax.GatherDimensionNumbers(offset_dims=(),
                                       collapsed_slice_dims=(0,),
                                       start_index_map=(0,)),
            (1,), mode=lax.GatherScatterMode.PROMISE_IN_BOUNDS) * V
        cp_ids.wait()

        def add_offsets(c):
            if sizes[c] == _L:
                return
            for i in range(offs[c] // _L, (offs[c] + sizes[c]) // _L):
                sl = pl.ds(i * _L, _L)
                idx_v[sl] = idx_v[sl] + gofs

        def gather(c):
            if sizes[c] == _L:
                idx = idx_v[pl.ds(offs[c], _L)] + gofs
                src = table_hbm.at[idx]
            else:
                src = table_hbm.at[idx_v.at[pl.ds(offs[c], sizes[c])]]
            return pltpu.async_copy(
                src, bufs[c % nbuf].at[pl.ds(0, sizes[c])],
                gsems[c % nbuf])

        def write(c):
            return pltpu.async_copy(
                bufs[c % nbuf].at[pl.ds(0, sizes[c])],
                out_hbm.at[pl.ds(base + offs[c], sizes[c]), 0],
                wsems[c % nbuf])

        gath = [None] * n
        wr = [None] * n
        head = min(nbuf, n)
        for c in range(head):
            add_offsets(c)
            gath[c] = gather(c)
        for c in range(head, n):
            add_offsets(c)
        drained = [False] * n
        for c in range(n):
            gath[c].wait()
            wr[c] = write(c)
            if c + nbuf < n:
                wr[c].wait()
                drained[c] = True
                gath[c + nbuf] = gather(c + nbuf)
        for c in range(n):
            if not drained[c]:
                wr[c].wait()

    return k


def kernel(tables, token_ids, group_idx):
    G, V, D = tables.shape
    B, S = token_ids.shape
    table_flat = tables.reshape(G * V, D)
    ids = token_ids.reshape(B * S)
    gid = jnp.asarray(group_idx, jnp.int32).reshape(1)
    out = _make_gather(B * S, V, D, 16, 6)(table_flat, ids, gid)
    return out.reshape(B, S, D) if S != 1 else out
